# baseline (speedup 1.0000x reference)
import functools
import math

import jax
import jax.numpy as jnp
from jax import lax
from jax.experimental import pallas as pl
from jax.experimental.pallas import tpu as pltpu
from jax.experimental.pallas import tpu_sc as plsc

EMBED_DIM = 32
SCALE = math.sqrt(float(EMBED_DIM))
NUM_CORES = 2
NUM_SUBCORES = 16
NUM_WORKERS = NUM_CORES * NUM_SUBCORES
LANES = 16
TABLE_PITCH = 128


def _make_sc_lookup(n_rows: int, n_cols: int):
    assert n_rows % NUM_WORKERS == 0
    rows_per_w = n_rows // NUM_WORKERS
    half = rows_per_w // 2
    n_chunks = 2 * n_cols
    pitch = half + 3

    mesh = plsc.VectorSubcoreMesh(
        core_axis_name="c", subcore_axis_name="s",
        num_cores=NUM_CORES, num_subcores=NUM_SUBCORES)

    @functools.partial(
        pl.kernel,
        out_type=jax.ShapeDtypeStruct((n_cols * EMBED_DIM, n_rows),
                                      jnp.float32),
        mesh=mesh,
        scratch_types=[
            pltpu.VMEM((n_cols, rows_per_w), jnp.int32),
            pltpu.VMEM((half, TABLE_PITCH), jnp.float32),
            pltpu.VMEM((half, TABLE_PITCH), jnp.float32),
            pltpu.VMEM((EMBED_DIM, pitch), jnp.float32),
            pltpu.VMEM((EMBED_DIM, pitch), jnp.float32),
            pltpu.SemaphoreType.DMA,
            pltpu.SemaphoreType.DMA,
            pltpu.SemaphoreType.DMA,
            pltpu.SemaphoreType.DMA,
        ],
        compiler_params=pltpu.CompilerParams(use_tc_tiling_on_sc=False,
                                             needs_layout_passes=False),
    )
    def sc_lookup(table_hbm, idxt_hbm, outt_hbm, idx_v, rows0, rows1, tb0,
                  tb1, gs0, gs1, os0, os1):
        wid = lax.axis_index("s") * NUM_CORES + lax.axis_index("c")
        col0 = wid * rows_per_w

        pltpu.sync_copy(idxt_hbm.at[:, pl.ds(col0, rows_per_w)], idx_v)

        lanes = [lax.broadcasted_iota(jnp.int32, (LANES,), 0) + h * LANES
                 for h in range(EMBED_DIM // LANES)]

        def offsets(g):
            return idx_v.at[g // 2, pl.ds((g % 2) * half, half)]

        def out_slice(g):
            return outt_hbm.at[pl.ds((g // 2) * EMBED_DIM, EMBED_DIM),
                               pl.ds(col0 + (g % 2) * half, half)]

        def start_gather(g, rows, sem):
            pltpu.async_copy(table_hbm.at[offsets(g)], rows, sem)

        def finish(g, rows, tb, gsem, osem, p):
            pltpu.make_async_copy(table_hbm.at[offsets(g)], rows,
                                  gsem).wait()

            @pl.when(p > 0)
            def _():
                pltpu.make_async_copy(tb.at[:, pl.ds(0, half)],
                                      out_slice(g - 2), osem).wait()

            def scale_t(i, carry):
                icol = jnp.full((LANES,), 0, jnp.int32) + i
                for h in range(EMBED_DIM // LANES):
                    v = rows[i, pl.ds(h * LANES, LANES)] * SCALE
                    plsc.store_scatter(tb, [lanes[h], icol], v)
                return carry

            lax.fori_loop(0, half, scale_t, 0, unroll=8)
            pltpu.async_copy(tb.at[:, pl.ds(0, half)], out_slice(g), osem)

        start_gather(0, rows0, gs0)

        def pair_body(p, carry):
            g0 = 2 * p
            start_gather(g0 + 1, rows1, gs1)
            finish(g0, rows0, tb0, gs0, os0, p)

            @pl.when(g0 + 2 < n_chunks)
            def _():
                start_gather(g0 + 2, rows0, gs0)

            finish(g0 + 1, rows1, tb1, gs1, os1, p)
            return carry

        lax.fori_loop(0, n_chunks // 2, pair_body, 0)
        pltpu.make_async_copy(tb0.at[:, pl.ds(0, half)],
                              out_slice(n_chunks - 2), os0).wait()
        pltpu.make_async_copy(tb1.at[:, pl.ds(0, half)],
                              out_slice(n_chunks - 1), os1).wait()

    return sc_lookup


def kernel(input, table):
    n_rows, n_cols = input.shape
    idxt = input.T.astype(jnp.int32)
    tpad = jnp.pad(table, ((0, 0), (0, TABLE_PITCH - EMBED_DIM)))
    out2d = _make_sc_lookup(n_rows, n_cols)(tpad, idxt)
    outt = out2d.reshape(n_cols, EMBED_DIM, n_rows)
    return jnp.transpose(outt, (2, 0, 1))
